# Initial kernel scaffold; baseline (speedup 1.0000x reference)
import jax
import jax.numpy as jnp
from jax.experimental import pallas as pl


def kernel(x1, edge_index1, batch1, x2, edge_index2, batch2, W1, b1, W2, b2, W3, b3, Wl, bl):
    raise NotImplementedError("write your pallas kernel here")


# speedup vs baseline: 15.6578x; 15.6578x over previous
import functools

import jax
import jax.numpy as jnp
from jax import lax
from jax.experimental import pallas as pl
from jax.experimental.pallas import tpu as pltpu
from jax.experimental.pallas import tpu_sc as plsc

N = 10000
GNUM = 16
D = 128
DOUT = 64
NC = 2
NS = 16
K = 128
NG = 10240
RPT = NG // NS
TRASH = NG - N
BLK = 2048


def _mesh():
  return plsc.VectorSubcoreMesh(core_axis_name="c", subcore_axis_name="s")


def _sc_degree(dst3):
  chunks = dst3.shape[1] // NS

  @functools.partial(
      pl.kernel,
      out_type=jax.ShapeDtypeStruct((NC, NG, 16), jnp.float32),
      mesh=_mesh(),
      scratch_types=[
          pltpu.VMEM_SHARED((NG, 16), jnp.float32),
          pltpu.VMEM((chunks, K), jnp.int32),
          pltpu.VMEM((K, 16), jnp.float32),
          pltpu.VMEM((RPT, 16), jnp.float32),
      ],
  )
  def kern(dst_h, deg_h, acc, didx, ones_v, zbuf):
    c = lax.axis_index("c")
    t = lax.axis_index("s")

    def fill_ones(i, _):
      ones_v[i, :] = jnp.ones((16,), jnp.float32)
      return 0
    lax.fori_loop(0, K, fill_ones, 0)

    def fill_zeros(i, _):
      zbuf[i, :] = jnp.zeros((16,), jnp.float32)
      return 0
    lax.fori_loop(0, RPT, fill_zeros, 0)

    pltpu.sync_copy(zbuf, acc.at[pl.ds(t * RPT, RPT)])
    pltpu.sync_copy(dst_h.at[c, pl.ds(t * chunks, chunks)], didx)
    plsc.subcore_barrier()

    def body(j, _):
      pltpu.sync_copy(ones_v, acc.at[didx.at[j]], add=True)
      return 0
    lax.fori_loop(0, chunks, body, 0)

    plsc.subcore_barrier()
    pltpu.sync_copy(acc.at[pl.ds(t * RPT, RPT)],
                    deg_h.at[c, pl.ds(t * RPT, RPT)])

  return kern(dst3)


IGC = 8


def _sc_scatter(gbuf, src3, dst3):
  chunks = src3.shape[1] // NS

  @functools.partial(
      pl.kernel,
      out_type=jax.ShapeDtypeStruct((NC, NG, D), jnp.float32),
      mesh=_mesh(),
      scratch_types=[
          pltpu.VMEM_SHARED((NG, D), jnp.float32),
          pltpu.VMEM((IGC, K), jnp.int32),
          pltpu.VMEM((IGC, K), jnp.int32),
          pltpu.VMEM((K, D), jnp.float32),
          pltpu.SemaphoreType.DMA,
      ],
  )
  def kern(g_h, src_h, dst_h, s_h, acc, sidx, didx, rb, gsem):
    c = lax.axis_index("c")
    t = lax.axis_index("s")

    def zf(i, _):
      rb[i // 8, pl.ds((i % 8) * 16, 16)] = jnp.zeros((16,), jnp.float32)
      return 0
    lax.fori_loop(0, K * D // 16, zf, 0)

    def zc(i, _):
      pltpu.sync_copy(rb, acc.at[pl.ds(t * RPT + i * K, K)])
      return 0
    lax.fori_loop(0, RPT // K, zc, 0)
    plsc.subcore_barrier()

    def outer(gi, _):
      base = t * chunks + gi * IGC
      pltpu.sync_copy(src_h.at[c, pl.ds(base, IGC)], sidx)
      pltpu.sync_copy(dst_h.at[c, pl.ds(base, IGC)], didx)

      def inner(j, _):
        pltpu.async_copy(g_h.at[sidx.at[j]], rb, gsem).wait()
        pltpu.sync_copy(rb, acc.at[didx.at[j]], add=True)
        return 0
      lax.fori_loop(0, IGC, inner, 0)
      return 0
    lax.fori_loop(0, chunks // IGC, outer, 0)

    plsc.subcore_barrier()
    pltpu.sync_copy(acc.at[pl.ds(t * RPT, RPT)],
                    s_h.at[c, pl.ds(t * RPT, RPT)])

  return kern(gbuf, src3, dst3)


def _dinv(deg_blk):
  return lax.rsqrt(deg_blk[:, 0:1] + 1.0)


def _tc_layer1(x, w1, deg16):
  def body(x_ref, w_ref, deg_ref, g_ref):
    dv = _dinv(deg_ref[...])
    h = jnp.dot(x_ref[...], w_ref[...], preferred_element_type=jnp.float32)
    g_ref[...] = dv * h

  grid = (NC * NG) // BLK
  return pl.pallas_call(
      body,
      grid=(grid,),
      in_specs=[
          pl.BlockSpec((BLK, D), lambda i: (i, 0)),
          pl.BlockSpec((D, D), lambda i: (0, 0)),
          pl.BlockSpec((BLK, 16), lambda i: (i, 0)),
      ],
      out_specs=pl.BlockSpec((BLK, D), lambda i: (i, 0)),
      out_shape=jax.ShapeDtypeStruct((NC * NG, D), jnp.float32),
  )(x, w1, deg16)


def _tc_layer_mid(s, g, deg16, b_prev, w_next):
  def body(s_ref, g_ref, deg_ref, b_ref, w_ref, o_ref):
    dv = _dinv(deg_ref[...])
    x = jnp.maximum(dv * (s_ref[...] + g_ref[...]) + b_ref[...], 0.0)
    h = jnp.dot(x, w_ref[...], preferred_element_type=jnp.float32)
    o_ref[...] = dv * h

  grid = (NC * NG) // BLK
  return pl.pallas_call(
      body,
      grid=(grid,),
      in_specs=[
          pl.BlockSpec((BLK, D), lambda i: (i, 0)),
          pl.BlockSpec((BLK, D), lambda i: (i, 0)),
          pl.BlockSpec((BLK, 16), lambda i: (i, 0)),
          pl.BlockSpec((1, D), lambda i: (0, 0)),
          pl.BlockSpec((D, D), lambda i: (0, 0)),
      ],
      out_specs=pl.BlockSpec((BLK, D), lambda i: (i, 0)),
      out_shape=jax.ShapeDtypeStruct((NC * NG, D), jnp.float32),
  )(s, g, deg16, b_prev, w_next)


def _tc_pool(s, g, deg16, b3, batch2d):
  blocks_per_graph = NG // BLK

  def body(s_ref, g_ref, deg_ref, b_ref, bat_ref, pool_ref, cnt_ref):
    i = pl.program_id(0)
    dv = _dinv(deg_ref[...])
    x = dv * (s_ref[...] + g_ref[...]) + b_ref[...]
    oh = (bat_ref[...] == lax.broadcasted_iota(jnp.int32, (1, GNUM), 1))
    oh = oh.astype(jnp.float32)
    pp = lax.dot_general(oh, x, (((0,), (0,)), ((), ())),
                         preferred_element_type=jnp.float32)
    cp = jnp.broadcast_to(jnp.sum(oh, axis=0)[:, None], (GNUM, D))

    @pl.when(i % blocks_per_graph == 0)
    def _():
      pool_ref[...] = pp[None]
      cnt_ref[...] = cp[None]

    @pl.when(i % blocks_per_graph != 0)
    def _():
      pool_ref[...] += pp[None]
      cnt_ref[...] += cp[None]

  grid = (NC * NG) // BLK
  return pl.pallas_call(
      body,
      grid=(grid,),
      in_specs=[
          pl.BlockSpec((BLK, D), lambda i: (i, 0)),
          pl.BlockSpec((BLK, D), lambda i: (i, 0)),
          pl.BlockSpec((BLK, 16), lambda i: (i, 0)),
          pl.BlockSpec((1, D), lambda i: (0, 0)),
          pl.BlockSpec((BLK, 1), lambda i: (i, 0)),
      ],
      out_specs=[
          pl.BlockSpec((1, GNUM, D), lambda i: (i // blocks_per_graph, 0, 0)),
          pl.BlockSpec((1, GNUM, D), lambda i: (i // blocks_per_graph, 0, 0)),
      ],
      out_shape=[
          jax.ShapeDtypeStruct((NC, GNUM, D), jnp.float32),
          jax.ShapeDtypeStruct((NC, GNUM, D), jnp.float32),
      ],
  )(s, g, deg16, b3, batch2d)


def _tc_dist(pooled, cnt, wl, bl):
  def body(p_ref, c_ref, w_ref, b_ref, o_ref):
    m = p_ref[...] / jnp.maximum(c_ref[...], 1.0)
    m2 = m.reshape(NC * GNUM, D)
    z = jnp.dot(m2, w_ref[...], preferred_element_type=jnp.float32)
    z = z + b_ref[...]
    diff = z[0:GNUM] - z[GNUM:2 * GNUM] + 1e-6
    o_ref[...] = jnp.sqrt(jnp.sum(diff * diff, axis=1))[None]

  return pl.pallas_call(
      body,
      out_shape=jax.ShapeDtypeStruct((1, GNUM), jnp.float32),
  )(pooled, cnt, wl, bl)


def kernel(x1, edge_index1, batch1, x2, edge_index2, batch2,
           W1, b1, W2, b2, W3, b3, Wl, bl):
  e = edge_index1.shape[1]
  chunks = -(-e // (NS * K))
  chunks = -(-chunks // 8) * 8
  ep = chunks * NS * K
  npad = ep - e

  pad_row = N + 4 + jnp.arange(npad, dtype=jnp.int32) % jnp.int32(TRASH - 8)
  zrow = jnp.zeros((TRASH, x1.shape[1]), x1.dtype)

  def prep_edges(ei, c):
    src = jnp.concatenate([ei[0], pad_row]) + jnp.int32(c * NG)
    dst = jnp.concatenate([ei[1], pad_row])
    return src.reshape(chunks * NS, K), dst.reshape(chunks * NS, K)

  s1, d1 = prep_edges(edge_index1, 0)
  s2, d2 = prep_edges(edge_index2, 1)
  src3 = jnp.stack([s1, s2])
  dst3 = jnp.stack([d1, d2])

  x = jnp.concatenate([x1, zrow, x2, zrow])
  padb = jnp.full((TRASH,), GNUM, jnp.int32)
  batch2d = jnp.concatenate([batch1, padb, batch2, padb]).reshape(NC * NG, 1)

  deg16 = _sc_degree(dst3).reshape(NC * NG, 16)

  g = _tc_layer1(x, W1, deg16)
  s = _sc_scatter(g, src3, dst3).reshape(NC * NG, D)
  g = _tc_layer_mid(s, g, deg16, b1.reshape(1, D), W2)
  s = _sc_scatter(g, src3, dst3).reshape(NC * NG, D)
  g = _tc_layer_mid(s, g, deg16, b2.reshape(1, D), W3)
  s = _sc_scatter(g, src3, dst3).reshape(NC * NG, D)

  pooled, cnt = _tc_pool(s, g, deg16, b3.reshape(1, D), batch2d)
  dist = _tc_dist(pooled, cnt, Wl, bl.reshape(1, DOUT))
  return dist.reshape(GNUM)


# speedup vs baseline: 21.6668x; 1.3838x over previous
import functools

import jax
import jax.numpy as jnp
from jax import lax
from jax.experimental import pallas as pl
from jax.experimental.pallas import tpu as pltpu
from jax.experimental.pallas import tpu_sc as plsc

N = 10000
GNUM = 16
D = 128
DOUT = 64
NC = 2
NS = 16
K = 128
NG = 10240
RPT = NG // NS
TRASH = NG - N
BLK = 2048


def _mesh():
  return plsc.VectorSubcoreMesh(core_axis_name="c", subcore_axis_name="s")


def _sc_degree(dst3):
  chunks = dst3.shape[1] // NS

  @functools.partial(
      pl.kernel,
      out_type=jax.ShapeDtypeStruct((NC, NG, 16), jnp.float32),
      mesh=_mesh(),
      scratch_types=[
          pltpu.VMEM_SHARED((NG, 16), jnp.float32),
          pltpu.VMEM((chunks, K), jnp.int32),
          pltpu.VMEM((K, 16), jnp.float32),
          pltpu.VMEM((RPT, 16), jnp.float32),
      ],
  )
  def kern(dst_h, deg_h, acc, didx, ones_v, zbuf):
    c = lax.axis_index("c")
    t = lax.axis_index("s")

    def fill_ones(i, _):
      ones_v[i, :] = jnp.ones((16,), jnp.float32)
      return 0
    lax.fori_loop(0, K, fill_ones, 0)

    def fill_zeros(i, _):
      zbuf[i, :] = jnp.zeros((16,), jnp.float32)
      return 0
    lax.fori_loop(0, RPT, fill_zeros, 0)

    pltpu.sync_copy(zbuf, acc.at[pl.ds(t * RPT, RPT)])
    pltpu.sync_copy(dst_h.at[c, pl.ds(t * chunks, chunks)], didx)
    plsc.subcore_barrier()

    def body(j, _):
      pltpu.sync_copy(ones_v, acc.at[didx.at[j]], add=True)
      return 0
    lax.fori_loop(0, chunks, body, 0)

    plsc.subcore_barrier()
    pltpu.sync_copy(acc.at[pl.ds(t * RPT, RPT)],
                    deg_h.at[c, pl.ds(t * RPT, RPT)])

  return kern(dst3)


IGC = 16


def _sc_scatter(gbuf, src3, dst3):
  chunks = src3.shape[1] // NS

  @functools.partial(
      pl.kernel,
      out_type=jax.ShapeDtypeStruct((NC, NG, D), jnp.float32),
      mesh=_mesh(),
      scratch_types=[
          pltpu.VMEM_SHARED((NG, D), jnp.float32),
          pltpu.VMEM((IGC, K), jnp.int32),
          pltpu.VMEM((IGC, K), jnp.int32),
          pltpu.VMEM((IGC, K), jnp.int32),
          pltpu.VMEM((IGC, K), jnp.int32),
          pltpu.VMEM((2, K, D), jnp.float32),
          pltpu.SemaphoreType.DMA,
          pltpu.SemaphoreType.DMA,
          pltpu.SemaphoreType.DMA,
      ],
  )
  def kern(g_h, src_h, dst_h, s_h, acc, sa, sb, da, db, rb,
           gsem0, gsem1, isem):
    c = lax.axis_index("c")
    t = lax.axis_index("s")

    def zf(i, _):
      rb[0, i // 8, pl.ds((i % 8) * 16, 16)] = jnp.zeros((16,), jnp.float32)
      return 0
    lax.fori_loop(0, K * D // 16, zf, 0)

    def zc(i, _):
      pltpu.sync_copy(rb.at[0], acc.at[pl.ds(t * RPT + i * K, K)])
      return 0
    lax.fori_loop(0, RPT // K, zc, 0)

    pltpu.sync_copy(src_h.at[c, pl.ds(t * chunks, IGC)], sa)
    pltpu.sync_copy(dst_h.at[c, pl.ds(t * chunks, IGC)], da)
    pltpu.async_copy(g_h.at[sa.at[0]], rb.at[0], gsem0)
    plsc.subcore_barrier()

    def body(j, _):
      gp = (j // IGC) % 2
      nj = j + 1
      ngp = (nj // IGC) % 2
      r = j % IGC
      nr = nj % IGC

      @pl.when(jnp.logical_and(r == 0, j + IGC < chunks))
      def _():
        base = t * chunks + IGC * (j // IGC + 1)

        @pl.when(gp == 0)
        def _():
          pltpu.async_copy(src_h.at[c, pl.ds(base, IGC)], sb, isem)
          pltpu.async_copy(dst_h.at[c, pl.ds(base, IGC)], db, isem)

        @pl.when(gp == 1)
        def _():
          pltpu.async_copy(src_h.at[c, pl.ds(base, IGC)], sa, isem)
          pltpu.async_copy(dst_h.at[c, pl.ds(base, IGC)], da, isem)

      @pl.when(j % 2 == 0)
      def _():
        pltpu.make_async_copy(g_h.at[sa.at[0]], rb.at[0], gsem0).wait()

      @pl.when(j % 2 == 1)
      def _():
        pltpu.make_async_copy(g_h.at[sa.at[0]], rb.at[1], gsem1).wait()

      @pl.when(jnp.logical_and(nj < chunks, nr == 0))
      def _():
        pltpu.make_async_copy(src_h.at[c, pl.ds(t * chunks, IGC)], sa,
                              isem).wait()
        pltpu.make_async_copy(dst_h.at[c, pl.ds(t * chunks, IGC)], da,
                              isem).wait()

      @pl.when(nj < chunks)
      def _():
        @pl.when(jnp.logical_and(nj % 2 == 0, ngp == 0))
        def _():
          pltpu.async_copy(g_h.at[sa.at[nr]], rb.at[0], gsem0)

        @pl.when(jnp.logical_and(nj % 2 == 0, ngp == 1))
        def _():
          pltpu.async_copy(g_h.at[sb.at[nr]], rb.at[0], gsem0)

        @pl.when(jnp.logical_and(nj % 2 == 1, ngp == 0))
        def _():
          pltpu.async_copy(g_h.at[sa.at[nr]], rb.at[1], gsem1)

        @pl.when(jnp.logical_and(nj % 2 == 1, ngp == 1))
        def _():
          pltpu.async_copy(g_h.at[sb.at[nr]], rb.at[1], gsem1)

      @pl.when(gp == 0)
      def _():
        pltpu.sync_copy(rb.at[j % 2], acc.at[da.at[r]], add=True)

      @pl.when(gp == 1)
      def _():
        pltpu.sync_copy(rb.at[j % 2], acc.at[db.at[r]], add=True)
      return 0
    lax.fori_loop(0, chunks, body, 0)

    plsc.subcore_barrier()
    pltpu.sync_copy(acc.at[pl.ds(t * RPT, RPT)],
                    s_h.at[c, pl.ds(t * RPT, RPT)])

  return kern(gbuf, src3, dst3)


def _dinv(deg_blk):
  return lax.rsqrt(deg_blk[:, 0:1] + 1.0)


def _tc_layer1(x, w1, deg16):
  def body(x_ref, w_ref, deg_ref, g_ref):
    dv = _dinv(deg_ref[...])
    h = jnp.dot(x_ref[...], w_ref[...], preferred_element_type=jnp.float32)
    g_ref[...] = dv * h

  grid = (NC * NG) // BLK
  return pl.pallas_call(
      body,
      grid=(grid,),
      in_specs=[
          pl.BlockSpec((BLK, D), lambda i: (i, 0)),
          pl.BlockSpec((D, D), lambda i: (0, 0)),
          pl.BlockSpec((BLK, 16), lambda i: (i, 0)),
      ],
      out_specs=pl.BlockSpec((BLK, D), lambda i: (i, 0)),
      out_shape=jax.ShapeDtypeStruct((NC * NG, D), jnp.float32),
  )(x, w1, deg16)


def _tc_layer_mid(s, g, deg16, b_prev, w_next):
  def body(s_ref, g_ref, deg_ref, b_ref, w_ref, o_ref):
    dv = _dinv(deg_ref[...])
    x = jnp.maximum(dv * (s_ref[...] + g_ref[...]) + b_ref[...], 0.0)
    h = jnp.dot(x, w_ref[...], preferred_element_type=jnp.float32)
    o_ref[...] = dv * h

  grid = (NC * NG) // BLK
  return pl.pallas_call(
      body,
      grid=(grid,),
      in_specs=[
          pl.BlockSpec((BLK, D), lambda i: (i, 0)),
          pl.BlockSpec((BLK, D), lambda i: (i, 0)),
          pl.BlockSpec((BLK, 16), lambda i: (i, 0)),
          pl.BlockSpec((1, D), lambda i: (0, 0)),
          pl.BlockSpec((D, D), lambda i: (0, 0)),
      ],
      out_specs=pl.BlockSpec((BLK, D), lambda i: (i, 0)),
      out_shape=jax.ShapeDtypeStruct((NC * NG, D), jnp.float32),
  )(s, g, deg16, b_prev, w_next)


def _tc_pool(s, g, deg16, b3, batch2d):
  blocks_per_graph = NG // BLK

  def body(s_ref, g_ref, deg_ref, b_ref, bat_ref, pool_ref, cnt_ref):
    i = pl.program_id(0)
    dv = _dinv(deg_ref[...])
    x = dv * (s_ref[...] + g_ref[...]) + b_ref[...]
    oh = (bat_ref[...] == lax.broadcasted_iota(jnp.int32, (1, GNUM), 1))
    oh = oh.astype(jnp.float32)
    pp = lax.dot_general(oh, x, (((0,), (0,)), ((), ())),
                         preferred_element_type=jnp.float32)
    cp = jnp.broadcast_to(jnp.sum(oh, axis=0)[:, None], (GNUM, D))

    @pl.when(i % blocks_per_graph == 0)
    def _():
      pool_ref[...] = pp[None]
      cnt_ref[...] = cp[None]

    @pl.when(i % blocks_per_graph != 0)
    def _():
      pool_ref[...] += pp[None]
      cnt_ref[...] += cp[None]

  grid = (NC * NG) // BLK
  return pl.pallas_call(
      body,
      grid=(grid,),
      in_specs=[
          pl.BlockSpec((BLK, D), lambda i: (i, 0)),
          pl.BlockSpec((BLK, D), lambda i: (i, 0)),
          pl.BlockSpec((BLK, 16), lambda i: (i, 0)),
          pl.BlockSpec((1, D), lambda i: (0, 0)),
          pl.BlockSpec((BLK, 1), lambda i: (i, 0)),
      ],
      out_specs=[
          pl.BlockSpec((1, GNUM, D), lambda i: (i // blocks_per_graph, 0, 0)),
          pl.BlockSpec((1, GNUM, D), lambda i: (i // blocks_per_graph, 0, 0)),
      ],
      out_shape=[
          jax.ShapeDtypeStruct((NC, GNUM, D), jnp.float32),
          jax.ShapeDtypeStruct((NC, GNUM, D), jnp.float32),
      ],
  )(s, g, deg16, b3, batch2d)


def _tc_dist(pooled, cnt, wl, bl):
  def body(p_ref, c_ref, w_ref, b_ref, o_ref):
    m = p_ref[...] / jnp.maximum(c_ref[...], 1.0)
    m2 = m.reshape(NC * GNUM, D)
    z = jnp.dot(m2, w_ref[...], preferred_element_type=jnp.float32)
    z = z + b_ref[...]
    diff = z[0:GNUM] - z[GNUM:2 * GNUM] + 1e-6
    o_ref[...] = jnp.sqrt(jnp.sum(diff * diff, axis=1))[None]

  return pl.pallas_call(
      body,
      out_shape=jax.ShapeDtypeStruct((1, GNUM), jnp.float32),
  )(pooled, cnt, wl, bl)


def kernel(x1, edge_index1, batch1, x2, edge_index2, batch2,
           W1, b1, W2, b2, W3, b3, Wl, bl):
  e = edge_index1.shape[1]
  chunks = -(-e // (NS * K))
  chunks = -(-chunks // 8) * 8
  ep = chunks * NS * K
  npad = ep - e

  pad_row = N + 4 + jnp.arange(npad, dtype=jnp.int32) % jnp.int32(TRASH - 8)
  zrow = jnp.zeros((TRASH, x1.shape[1]), x1.dtype)

  def prep_edges(ei, c):
    src = jnp.concatenate([ei[0], pad_row]) + jnp.int32(c * NG)
    dst = jnp.concatenate([ei[1], pad_row])
    return src.reshape(chunks * NS, K), dst.reshape(chunks * NS, K)

  s1, d1 = prep_edges(edge_index1, 0)
  s2, d2 = prep_edges(edge_index2, 1)
  src3 = jnp.stack([s1, s2])
  dst3 = jnp.stack([d1, d2])

  x = jnp.concatenate([x1, zrow, x2, zrow])
  padb = jnp.full((TRASH,), GNUM, jnp.int32)
  batch2d = jnp.concatenate([batch1, padb, batch2, padb]).reshape(NC * NG, 1)

  deg16 = _sc_degree(dst3).reshape(NC * NG, 16)

  g = _tc_layer1(x, W1, deg16)
  s = _sc_scatter(g, src3, dst3).reshape(NC * NG, D)
  g = _tc_layer_mid(s, g, deg16, b1.reshape(1, D), W2)
  s = _sc_scatter(g, src3, dst3).reshape(NC * NG, D)
  g = _tc_layer_mid(s, g, deg16, b2.reshape(1, D), W3)
  s = _sc_scatter(g, src3, dst3).reshape(NC * NG, D)

  pooled, cnt = _tc_pool(s, g, deg16, b3.reshape(1, D), batch2d)
  dist = _tc_dist(pooled, cnt, Wl, bl.reshape(1, DOUT))
  return dist.reshape(GNUM)
